# baseline (speedup 1.0000x reference)
import functools

import jax
import jax.numpy as jnp
from jax import lax
from jax.experimental import pallas as pl
from jax.experimental.pallas import tpu as pltpu
from jax.experimental.pallas import tpu_sc as plsc

_B, _T, _D = 4096, 200, 64
_W = _T // 4
_NC, _NS, _L = 2, 16, 16
_NW = _NC * _NS
_RPW = _B // _NW
_GROUPS = _RPW // _L


@functools.partial(
    pl.kernel,
    out_type=jax.ShapeDtypeStruct((_B, _D), jnp.float32),
    mesh=plsc.VectorSubcoreMesh(core_axis_name="c", subcore_axis_name="s"),
    scratch_types=[
        pltpu.VMEM((_RPW * _W,), jnp.int32),
        pltpu.VMEM((_RPW,), jnp.int32),
        pltpu.VMEM((_RPW, _D), jnp.float32),
        pltpu.SemaphoreType.DMA,
    ],
    compiler_params=pltpu.CompilerParams(
        needs_layout_passes=False, use_tc_tiling_on_sc=False
    ),
)
def _last_pool_sc(x_hbm, mw_hbm, out_hbm, mwbuf, idxbuf, rowsbuf, sem):
    wid = lax.axis_index("s") * _NC + lax.axis_index("c")
    base = wid * _RPW
    pltpu.sync_copy(mw_hbm.at[pl.ds(base * _W, _RPW * _W)], mwbuf)
    lanes = lax.iota(jnp.int32, _L)
    for g in range(_GROUPS):
        row = jnp.full((_L,), g * _L, jnp.int32) + lanes
        word0 = row * _W

        def body(j, acc, word0=word0):
            w = plsc.load_gather(mwbuf, [word0 + j])
            return acc + w

        acc = lax.fori_loop(0, _W, body, jnp.zeros((_L,), jnp.int32))
        s = (
            (acc & 0xFF)
            + ((acc >> 8) & 0xFF)
            + ((acc >> 16) & 0xFF)
            + ((acc >> 24) & 0xFF)
        )
        t = jnp.minimum(s, _T - 1) - 1
        t = jnp.where(t < 0, _T - 1, t)
        idxbuf[pl.ds(g * _L, _L)] = (base + row) * _T + t
    pltpu.async_copy(x_hbm.at[idxbuf], rowsbuf, sem).wait()
    pltpu.sync_copy(rowsbuf, out_hbm.at[pl.ds(base, _RPW)])


def kernel(x, mask):
    mask_words = lax.bitcast_convert_type(
        mask.astype(jnp.uint8).reshape(_B, _W, 4), jnp.int32
    )
    return _last_pool_sc(x.reshape(_B * _T, _D), mask_words.reshape(_B * _W))


# speedup vs baseline: 9.1096x; 9.1096x over previous
import functools

import jax
import jax.numpy as jnp
from jax import lax
from jax.experimental import pallas as pl
from jax.experimental.pallas import tpu as pltpu
from jax.experimental.pallas import tpu_sc as plsc

_B, _T, _D = 4096, 200, 64
_L = 16
_NC, _NS = 2, 16
_NW = _NC * _NS
_WPT = 128 // 4
_GR = _T * _D * _B // _L


@functools.partial(
    pl.kernel,
    out_type=jax.ShapeDtypeStruct((8, 32, 8, 128), jnp.float32),
    mesh=plsc.VectorSubcoreMesh(core_axis_name="c", subcore_axis_name="s"),
    scratch_types=[
        pltpu.VMEM((_T, _WPT), jnp.int32),
        pltpu.VMEM((_WPT,), jnp.int32),
        pltpu.VMEM((8, 128), jnp.int32),
        pltpu.VMEM((1024, _L), jnp.float32),
        pltpu.VMEM((8, 1, 8, 128), jnp.float32),
        pltpu.SemaphoreType.DMA,
    ],
    compiler_params=pltpu.CompilerParams(
        needs_layout_passes=False, use_tc_tiling_on_sc=False
    ),
)
def _last_pool_sc(a16_hbm, mwt_hbm, out_hbm, mbuf, accbuf, idxbuf, gbuf, obuf, sem):
    wid = lax.axis_index("s") * _NC + lax.axis_index("c")
    pltpu.sync_copy(mwt_hbm.at[:, pl.ds(wid * _WPT, _WPT)], mbuf)
    lanes = lax.iota(jnp.int32, _L)
    zero = jnp.zeros((_L,), jnp.int32)

    def mbody(t, accs):
        a0, a1 = accs
        row = mbuf.at[t]
        return a0 + row[pl.ds(0, _L)], a1 + row[pl.ds(_L, _L)]

    a0, a1 = lax.fori_loop(0, _T, mbody, (zero, zero))
    accbuf[pl.ds(0, _L)] = a0
    accbuf[pl.ds(_L, _L)] = a1

    for g in range(8):
        w = plsc.load_gather(accbuf, [g * 4 + (lanes >> 2)])
        s = (w >> ((lanes & 3) * 8)) & 0xFF
        t = jnp.minimum(s, _T - 1) - 1
        t = jnp.where(t < 0, _T - 1, t)
        base = t * (_D * _B // _L) + wid * 64 + g
        for k in range(8):
            for dl in range(8):
                idxbuf.at[k][pl.ds(dl * _L, _L)] = base + k * 2048 + dl * 8
        copies = [
            pltpu.async_copy(
                a16_hbm.at[idxbuf.at[k]], gbuf.at[pl.ds(k * 128, 128)], sem
            )
            for k in range(8)
        ]
        for c in copies:
            c.wait()
        for d in range(_D):
            val = plsc.load_gather(gbuf, [d * _L + lanes, lanes])
            obuf.at[d >> 3].at[0].at[d & 7][pl.ds(g * _L, _L)] = val

    pltpu.sync_copy(obuf, out_hbm.at[:, pl.ds(wid, 1)])


def kernel(x, mask):
    a16 = (
        x.reshape(32, 128, _T, 8, 8)
        .transpose(2, 3, 0, 4, 1)
        .reshape(_GR, _L)
    )
    mask_words = lax.bitcast_convert_type(
        mask.T.astype(jnp.uint8).reshape(_T, _B // 4, 4), jnp.int32
    )
    out_t = _last_pool_sc(a16, mask_words)
    return out_t.transpose(1, 3, 0, 2).reshape(_B, _D)


# speedup vs baseline: 12.9166x; 1.4179x over previous
import functools

import jax
import jax.numpy as jnp
from jax import lax
from jax.experimental import pallas as pl
from jax.experimental.pallas import tpu as pltpu
from jax.experimental.pallas import tpu_sc as plsc

_B, _T, _D = 4096, 200, 64
_L = 16
_NC, _NS = 2, 16
_NW = _NC * _NS
_TT = _T // 8
_GR = _T * _D * _B // _L
_TS = _D * _B // _L


@functools.partial(
    pl.kernel,
    out_type=jax.ShapeDtypeStruct((8, 32, 8, 128), jnp.float32),
    mesh=plsc.VectorSubcoreMesh(core_axis_name="c", subcore_axis_name="s"),
    scratch_types=[
        pltpu.VMEM((_TT, 8, 128), jnp.int32),
        pltpu.VMEM((2, 8, 128), jnp.int32),
        pltpu.VMEM((2, 1024, _L), jnp.float32),
        pltpu.VMEM((8, 1, 8, 128), jnp.float32),
        pltpu.SemaphoreType.DMA,
        pltpu.SemaphoreType.DMA,
    ],
    compiler_params=pltpu.CompilerParams(
        needs_layout_passes=False, use_tc_tiling_on_sc=False
    ),
)
def _last_pool_sc(a16_hbm, mw_hbm, out_hbm, mbuf, idxbuf, gbuf, obuf, sem0, sem1):
    wid = lax.axis_index("s") * _NC + lax.axis_index("c")
    pltpu.sync_copy(mw_hbm.at[:, wid], mbuf)
    zero = jnp.zeros((_L,), jnp.int32)

    def mbody(tt, accs):
        row = mbuf.at[tt]
        out = list(accs)
        for tr in range(8):
            r = row.at[tr]
            for g in range(8):
                out[g] = out[g] + r[pl.ds(g * _L, _L)]
        return tuple(out)

    accs = lax.fori_loop(0, _TT, mbody, (zero,) * 8)

    bases = []
    for g in range(8):
        s = accs[g]
        t = jnp.minimum(s, _T - 1) - 1
        t = jnp.where(t < 0, _T - 1, t)
        bases.append(t * _TS + wid * 64 + g)

    def fill_idx(g, buf):
        for k in range(8):
            for dl in range(8):
                buf.at[k][pl.ds(dl * _L, _L)] = bases[g] + k * 2048 + dl * 8

    def fire(half):
        s = sem0 if half == 0 else sem1
        return [
            pltpu.async_copy(
                a16_hbm.at[idxbuf.at[half].at[k]],
                gbuf.at[half].at[pl.ds(k * 128, 128)],
                s,
            )
            for k in range(8)
        ]

    lanes = lax.iota(jnp.int32, _L)
    fill_idx(0, idxbuf.at[0])
    copies = fire(0)
    for g in range(8):
        half = g & 1
        if g < 7:
            fill_idx(g + 1, idxbuf.at[1 - half])
            next_copies = fire(1 - half)
        for c in copies:
            c.wait()
        for d in range(_D):
            val = plsc.load_gather(gbuf.at[half], [d * _L + lanes, lanes])
            obuf.at[d >> 3].at[0].at[d & 7][pl.ds(g * _L, _L)] = val
        if g < 7:
            copies = next_copies

    pltpu.sync_copy(obuf, out_hbm.at[:, pl.ds(wid, 1)])


def kernel(x, mask):
    a16 = (
        x.reshape(32, 128, _T, 8, 8)
        .transpose(2, 3, 0, 4, 1)
        .reshape(_GR, _L)
    )
    mask_words = (
        mask.astype(jnp.int32)
        .reshape(32, 128, _TT, 8)
        .transpose(2, 0, 3, 1)
    )
    out_t = _last_pool_sc(a16, mask_words)
    return out_t.transpose(1, 3, 0, 2).reshape(_B, _D)


# speedup vs baseline: 14.2660x; 1.1045x over previous
import functools

import jax
import jax.numpy as jnp
from jax import lax
from jax.experimental import pallas as pl
from jax.experimental.pallas import tpu as pltpu
from jax.experimental.pallas import tpu_sc as plsc

_B, _T, _D = 4096, 200, 64
_L = 16
_NC, _NS = 2, 16
_NW = _NC * _NS
_TT = _T // 8
_GR = _T * _D * _B // _L
_TS = _D * _B // _L


@functools.partial(
    pl.kernel,
    out_type=jax.ShapeDtypeStruct((8, 32, 8, 128), jnp.float32),
    mesh=plsc.VectorSubcoreMesh(core_axis_name="c", subcore_axis_name="s"),
    scratch_types=[
        pltpu.VMEM((_TT, 8, 128), jnp.int32),
        pltpu.VMEM((128,), jnp.int32),
        pltpu.VMEM((2, 8, 128), jnp.int32),
        pltpu.VMEM((2, 1024, _L), jnp.float32),
        pltpu.VMEM((8, 1, 8, 128), jnp.float32),
        pltpu.SemaphoreType.DMA,
        pltpu.SemaphoreType.DMA,
    ],
    compiler_params=pltpu.CompilerParams(
        needs_layout_passes=False, use_tc_tiling_on_sc=False
    ),
)
def _last_pool_sc(a16_hbm, mw_hbm, out_hbm, mbuf, bbuf, idxbuf, gbuf, obuf, s0, s1):
    wid = lax.axis_index("s") * _NC + lax.axis_index("c")
    pltpu.sync_copy(mw_hbm.at[:, wid], mbuf)
    zero = jnp.zeros((_L,), jnp.int32)

    def mbody(tt, accs):
        row = mbuf.at[tt]
        out = list(accs)
        for tr in range(8):
            r = row.at[tr]
            for g in range(8):
                out[g] = out[g] + r[pl.ds(g * _L, _L)]
        return tuple(out)

    accs = lax.fori_loop(0, _TT, mbody, (zero,) * 8)
    for g in range(8):
        t = jnp.minimum(accs[g], _T - 1) - 1
        t = jnp.where(t < 0, _T - 1, t)
        bbuf[pl.ds(g * _L, _L)] = t * _TS + wid * 64 + g

    def fill_idx(g, h):
        base = bbuf[pl.ds(g * _L, _L)]

        def kbody(k, _):
            row = idxbuf.at[h].at[k]
            for dl in range(8):
                row[pl.ds(dl * _L, _L)] = base + k * 2048 + dl * 8
            return 0

        lax.fori_loop(0, 8, kbody, 0)

    def fire(h, sem):
        def kbody(k, _):
            pltpu.async_copy(
                a16_hbm.at[idxbuf.at[h].at[k]],
                gbuf.at[h].at[pl.ds(k * 128, 128)],
                sem,
            )
            return 0

        lax.fori_loop(0, 8, kbody, 0)

    def drain(h, sem):
        def kbody(k, _):
            pltpu.make_async_copy(
                a16_hbm.at[idxbuf.at[h].at[k]],
                gbuf.at[h].at[pl.ds(k * 128, 128)],
                sem,
            ).wait()
            return 0

        lax.fori_loop(0, 8, kbody, 0)

    lanes = lax.iota(jnp.int32, _L)

    def extract(g, h):
        src = gbuf.at[h]

        def jbody(j, _):
            for dl in range(8):
                val = plsc.load_gather(src, [j * 128 + dl * _L + lanes, lanes])
                obuf.at[j].at[0].at[dl][pl.ds(g * _L, _L)] = val
            return 0

        lax.fori_loop(0, 8, jbody, 0)

    fill_idx(0, 0)
    fire(0, s0)

    def step(st, _):
        g0 = 2 * st
        fill_idx(g0 + 1, 1)
        fire(1, s1)
        drain(0, s0)
        extract(g0, 0)

        @pl.when(st < 3)
        def _():
            fill_idx(g0 + 2, 0)
            fire(0, s0)

        drain(1, s1)
        extract(g0 + 1, 1)
        return 0

    lax.fori_loop(0, 4, step, 0)
    pltpu.sync_copy(obuf, out_hbm.at[:, pl.ds(wid, 1)])


def kernel(x, mask):
    a16 = (
        x.reshape(32, 128, _T, 8, 8)
        .transpose(2, 3, 0, 4, 1)
        .reshape(_GR, _L)
    )
    mask_words = (
        mask.astype(jnp.int32)
        .reshape(32, 128, _TT, 8)
        .transpose(2, 0, 3, 1)
    )
    out_t = _last_pool_sc(a16, mask_words)
    return out_t.transpose(1, 3, 0, 2).reshape(_B, _D)


# speedup vs baseline: 14.6073x; 1.0239x over previous
import functools

import jax
import jax.numpy as jnp
from jax import lax
from jax.experimental import pallas as pl
from jax.experimental.pallas import tpu as pltpu
from jax.experimental.pallas import tpu_sc as plsc

_B, _T, _D = 4096, 200, 64
_L = 16
_NC, _NS = 2, 16
_NW = _NC * _NS
_TT = _T // 8
_GR = _T * _D * _B // _L
_TS = _D * _B // _L


@functools.partial(
    pl.kernel,
    out_type=jax.ShapeDtypeStruct((8, 32, 8, 128), jnp.float32),
    mesh=plsc.VectorSubcoreMesh(core_axis_name="c", subcore_axis_name="s"),
    scratch_types=[
        pltpu.VMEM((_TT, 8, 128), jnp.int32),
        pltpu.VMEM((2, 8, 128), jnp.int32),
        pltpu.VMEM((2, 1024, _L), jnp.float32),
        pltpu.VMEM((8, 1, 8, 128), jnp.float32),
        pltpu.SemaphoreType.DMA,
        pltpu.SemaphoreType.DMA,
    ],
    compiler_params=pltpu.CompilerParams(
        needs_layout_passes=False, use_tc_tiling_on_sc=False
    ),
)
def _last_pool_sc(a16_hbm, mw_hbm, out_hbm, mbuf, idxbuf, gbuf, obuf, s0, s1):
    wid = lax.axis_index("s") * _NC + lax.axis_index("c")
    pltpu.sync_copy(mw_hbm.at[:, wid], mbuf)
    zero = jnp.zeros((_L,), jnp.int32)
    lanes = lax.iota(jnp.int32, _L)

    def fill_idx(g, h):
        def tbody(tt, acc):
            row = mbuf.at[tt]
            for tr in range(8):
                acc = acc + row.at[tr][pl.ds(g * _L, _L)]
            return acc

        s = lax.fori_loop(0, _TT, tbody, zero)
        t = jnp.minimum(s, _T - 1) - 1
        t = jnp.where(t < 0, _T - 1, t)
        base = t * _TS + wid * 64 + g

        def kbody(k, _):
            row = idxbuf.at[h].at[k]
            for dl in range(8):
                row[pl.ds(dl * _L, _L)] = base + k * 2048 + dl * 8
            return 0

        lax.fori_loop(0, 8, kbody, 0)

    def fire(h, sem):
        def kbody(k, _):
            pltpu.async_copy(
                a16_hbm.at[idxbuf.at[h].at[k]],
                gbuf.at[h].at[pl.ds(k * 128, 128)],
                sem,
            )
            return 0

        lax.fori_loop(0, 8, kbody, 0)

    def drain(h, sem):
        def kbody(k, _):
            pltpu.make_async_copy(
                a16_hbm.at[idxbuf.at[h].at[k]],
                gbuf.at[h].at[pl.ds(k * 128, 128)],
                sem,
            ).wait()
            return 0

        lax.fori_loop(0, 8, kbody, 0)

    def extract(g, h):
        src = gbuf.at[h]

        def jbody(j, _):
            for dl in range(8):
                val = plsc.load_gather(src, [j * 128 + dl * _L + lanes, lanes])
                obuf.at[j].at[0].at[dl][pl.ds(g * _L, _L)] = val
            return 0

        lax.fori_loop(0, 8, jbody, 0)

    fill_idx(0, 0)
    fire(0, s0)

    def step(g, _):
        h = g & 1

        @pl.when(g < 7)
        def _():
            fill_idx(g + 1, 1 - h)

            @pl.when(h == 0)
            def _():
                fire(1, s1)

            @pl.when(h == 1)
            def _():
                fire(0, s0)

        @pl.when(h == 0)
        def _():
            drain(0, s0)

        @pl.when(h == 1)
        def _():
            drain(1, s1)

        extract(g, h)
        return 0

    lax.fori_loop(0, 8, step, 0)
    pltpu.sync_copy(obuf, out_hbm.at[:, pl.ds(wid, 1)])


def kernel(x, mask):
    a16 = (
        x.reshape(32, 128, _T, 8, 8)
        .transpose(2, 3, 0, 4, 1)
        .reshape(_GR, _L)
    )
    mask_words = (
        mask.astype(jnp.int32)
        .reshape(32, 128, _TT, 8)
        .transpose(2, 0, 3, 1)
    )
    out_t = _last_pool_sc(a16, mask_words)
    return out_t.transpose(1, 3, 0, 2).reshape(_B, _D)


# speedup vs baseline: 14.8528x; 1.0168x over previous
import functools

import jax
import jax.numpy as jnp
from jax import lax
from jax.experimental import pallas as pl
from jax.experimental.pallas import tpu as pltpu
from jax.experimental.pallas import tpu_sc as plsc

_B, _T, _D = 4096, 200, 64
_L = 16
_NC, _NS = 2, 16
_NW = _NC * _NS
_TT = _T // 8
_GR = _T * _D * _B // _L
_TS = _D * _B // _L


@functools.partial(
    pl.kernel,
    out_type=jax.ShapeDtypeStruct((8, 32, 8, 128), jnp.float32),
    mesh=plsc.VectorSubcoreMesh(core_axis_name="c", subcore_axis_name="s"),
    scratch_types=[
        pltpu.VMEM((_TT, 8, 128), jnp.int32),
        pltpu.VMEM((4, 8, 128), jnp.int32),
        pltpu.VMEM((4, 1024, _L), jnp.float32),
        pltpu.VMEM((8, 1, 8, 128), jnp.float32),
        pltpu.SemaphoreType.DMA,
        pltpu.SemaphoreType.DMA,
        pltpu.SemaphoreType.DMA,
        pltpu.SemaphoreType.DMA,
    ],
    compiler_params=pltpu.CompilerParams(
        needs_layout_passes=False, use_tc_tiling_on_sc=False
    ),
)
def _last_pool_sc(
    a16_hbm, mw_hbm, out_hbm, mbuf, idxbuf, gbuf, obuf, s0, s1, s2, s3
):
    wid = lax.axis_index("s") * _NC + lax.axis_index("c")
    pltpu.sync_copy(mw_hbm.at[:, wid], mbuf)
    zero = jnp.zeros((_L,), jnp.int32)
    lanes = lax.iota(jnp.int32, _L)

    def fill_idx(g, h):
        def tbody(tt, acc):
            row = mbuf.at[tt]
            for tr in range(8):
                acc = acc + row.at[tr][pl.ds(g * _L, _L)]
            return acc

        s = lax.fori_loop(0, _TT, tbody, zero)
        t = jnp.minimum(s, _T - 1) - 1
        t = jnp.where(t < 0, _T - 1, t)
        base = t * _TS + wid * 64 + g

        def kbody(k, _):
            row = idxbuf.at[h].at[k]
            for dl in range(8):
                row[pl.ds(dl * _L, _L)] = base + k * 2048 + dl * 8
            return 0

        lax.fori_loop(0, 8, kbody, 0)

    def fire(h, sem):
        def kbody(k, _):
            pltpu.async_copy(
                a16_hbm.at[idxbuf.at[h].at[k]],
                gbuf.at[h].at[pl.ds(k * 128, 128)],
                sem,
            )
            return 0

        lax.fori_loop(0, 8, kbody, 0)

    def drain(h, sem):
        def kbody(k, _):
            pltpu.make_async_copy(
                a16_hbm.at[idxbuf.at[h].at[k]],
                gbuf.at[h].at[pl.ds(k * 128, 128)],
                sem,
            ).wait()
            return 0

        lax.fori_loop(0, 8, kbody, 0)

    def extract(g, h):
        src = gbuf.at[h]

        def jbody(j, _):
            for dl in range(8):
                val = plsc.load_gather(src, [j * 128 + dl * _L + lanes, lanes])
                obuf.at[j].at[0].at[dl][pl.ds(g * _L, _L)] = val
            return 0

        lax.fori_loop(0, 8, jbody, 0)

    sems = (s0, s1, s2, s3)
    for g in range(3):
        fill_idx(g, g)
        fire(g, sems[g])

    def step(g, _):
        h = g & 3
        for hh in range(4):
            @pl.when(h == hh)
            def _(hh=hh):
                drain(hh, sems[hh])

        extract(g, h)

        @pl.when(g < 5)
        def _():
            hn = (g + 3) & 3
            fill_idx(g + 3, hn)
            for hh in range(4):
                @pl.when(hn == hh)
                def _(hh=hh):
                    fire(hh, sems[hh])

        return 0

    lax.fori_loop(0, 8, step, 0)
    pltpu.sync_copy(obuf, out_hbm.at[:, pl.ds(wid, 1)])


def kernel(x, mask):
    a16 = (
        x.reshape(32, 128, _T, 8, 8)
        .transpose(2, 3, 0, 4, 1)
        .reshape(_GR, _L)
    )
    mask_words = (
        mask.astype(jnp.int32)
        .reshape(32, 128, _TT, 8)
        .transpose(2, 0, 3, 1)
    )
    out_t = _last_pool_sc(a16, mask_words)
    return out_t.transpose(1, 3, 0, 2).reshape(_B, _D)
